# baseline (speedup 1.0000x reference)
import functools

import jax
import jax.numpy as jnp
from jax import lax
from jax.experimental import pallas as pl
from jax.experimental.pallas import tpu as pltpu
from jax.experimental.pallas import tpu_sc as plsc

N, D, E = 10000, 128, 320000
NC, NS = 2, 16
DH = D // NC
EPT = E // NS
CH = 80
NCHUNK = EPT // CH
NP = 10240
RPT = NP // NS
BN_EPS = 1e-5


def _make_agg():
    mesh = plsc.VectorSubcoreMesh(core_axis_name="c", subcore_axis_name="s")

    @functools.partial(
        pl.kernel,
        mesh=mesh,
        compiler_params=pltpu.CompilerParams(use_tc_tiling_on_sc=False),
        out_type=jax.ShapeDtypeStruct((NP, D), jnp.float32),
        scratch_types=[
            pltpu.VMEM((NCHUNK, CH), jnp.int32),
            pltpu.VMEM((NCHUNK, CH), jnp.int32),
            pltpu.VMEM((NCHUNK, CH), jnp.int32),
            pltpu.VMEM((CH, DH), jnp.float32),
            pltpu.VMEM((CH, DH), jnp.float32),
            pltpu.VMEM_SHARED((NP, DH), jnp.float32),
            pltpu.SemaphoreType.DMA,
            pltpu.SemaphoreType.DMA,
            pltpu.SemaphoreType.DMA,
            pltpu.SemaphoreType.DMA,
        ],
    )
    def agg(x_hbm, src_hbm, dst_hbm, zero_hbm, out_hbm,
            raw_v, src_v, dst_v, rows0_v, rows1_v, acc_sh,
            gsem0, gsem1, ssem0, ssem1):
        c = lax.axis_index("c")
        s = lax.axis_index("s")

        pltpu.sync_copy(zero_hbm, acc_sh.at[pl.ds(s * RPT, RPT)])

        pltpu.sync_copy(src_hbm.at[s], raw_v)
        pltpu.sync_copy(dst_hbm.at[s], dst_v)
        plsc.subcore_barrier()

        def prep(m):
            for t in range(CH // 16):
                v = raw_v[m, pl.ds(16 * t, 16)]
                src_v[m, pl.ds(16 * t, 16)] = v + v + c

        def body(k, carry):
            a = 2 * k
            not_last = k < NCHUNK // 2 - 1

            @pl.when(k >= 0)
            def _():
                pltpu.make_async_copy(x_hbm.at[src_v.at[0]], rows0_v,
                                      gsem0).wait()
                pltpu.async_copy(rows0_v, acc_sh.at[dst_v.at[a]], ssem0,
                                 add=True)
                pltpu.make_async_copy(x_hbm.at[src_v.at[0]], rows1_v,
                                      gsem1).wait()
                pltpu.async_copy(rows1_v, acc_sh.at[dst_v.at[a + 1]], ssem1,
                                 add=True)
                pltpu.make_async_copy(rows0_v, acc_sh.at[dst_v.at[0]],
                                      ssem0).wait()

            @pl.when(not_last)
            def _():
                prep(a + 2)
                pltpu.async_copy(x_hbm.at[src_v.at[a + 2]], rows0_v, gsem0)

            @pl.when(k >= 0)
            def _():
                pltpu.make_async_copy(rows1_v, acc_sh.at[dst_v.at[0]],
                                      ssem1).wait()

            @pl.when(not_last)
            def _():
                prep(a + 3)
                pltpu.async_copy(x_hbm.at[src_v.at[a + 3]], rows1_v, gsem1)

            return carry

        lax.fori_loop(-1, NCHUNK // 2, body, 0)
        plsc.subcore_barrier()

        pltpu.sync_copy(acc_sh.at[pl.ds(s * RPT, RPT)],
                        out_hbm.at[pl.ds(s * RPT, RPT), pl.ds(c * DH, DH)])

    return agg


_agg_cache = []


def _agg(*args):
    if not _agg_cache:
        _agg_cache.append(_make_agg())
    return _agg_cache[0](*args)


def _mlp1_body(parts_ref, x_ref, w_ref, b_ref, gm_ref, bt_ref, out_ref):
    aggv = parts_ref[:N] + x_ref[...]
    y = jnp.dot(aggv, w_ref[...], preferred_element_type=jnp.float32)
    y = y + b_ref[...]
    mu = jnp.mean(y, axis=0, keepdims=True)
    var = jnp.mean((y - mu) ** 2, axis=0, keepdims=True)
    yn = gm_ref[...] * (y - mu) * lax.rsqrt(var + BN_EPS) + bt_ref[...]
    out_ref[...] = jnp.maximum(yn, 0.0)


def _mlp2_body(parts_ref, x_ref, wa_ref, ba_ref, gm_ref, bt_ref, wb_ref,
               bb_ref, out_ref):
    aggv = parts_ref[:N] + x_ref[...]
    y = jnp.dot(aggv, wa_ref[...], preferred_element_type=jnp.float32)
    y = y + ba_ref[...]
    mu = jnp.mean(y, axis=0, keepdims=True)
    var = jnp.mean((y - mu) ** 2, axis=0, keepdims=True)
    z = jnp.maximum(gm_ref[...] * (y - mu) * lax.rsqrt(var + BN_EPS)
                    + bt_ref[...], 0.0)
    h2 = jnp.dot(z, wb_ref[...], preferred_element_type=jnp.float32)
    out_ref[...] = jnp.maximum(h2 + bb_ref[...], 0.0)


def _mlp1(parts, x, W1, b1, g1, be1):
    return pl.pallas_call(
        _mlp1_body,
        out_shape=jax.ShapeDtypeStruct((N, D), jnp.float32),
    )(parts, x, W1, b1.reshape(1, D), g1.reshape(1, D), be1.reshape(1, D))


def _mlp2(parts, x, W2a, b2a, g2, be2, W2b, b2b):
    return pl.pallas_call(
        _mlp2_body,
        out_shape=jax.ShapeDtypeStruct((N, D), jnp.float32),
    )(parts, x, W2a, b2a.reshape(1, D), g2.reshape(1, D), be2.reshape(1, D),
      W2b, b2b.reshape(1, D))


def kernel(g, h, W1, b1, g1, be1, W2a, b2a, g2, be2, W2b, b2b):
    src = g[0].astype(jnp.int32).reshape(NS, NCHUNK, CH)
    dst = g[1].astype(jnp.int32).reshape(NS, NCHUNK, CH)
    zeros = jnp.zeros((RPT, DH), jnp.float32)
    parts1 = _agg(h.reshape(N * NC, DH), src, dst, zeros)
    h1 = _mlp1(parts1, h, W1, b1, g1, be1)
    parts2 = _agg(h1.reshape(N * NC, DH), src, dst, zeros)
    return _mlp2(parts2, h1, W2a, b2a, g2, be2, W2b, b2b)


# speedup vs baseline: 1.6314x; 1.6314x over previous
import functools

import jax
import jax.numpy as jnp
from jax import lax
from jax.experimental import pallas as pl
from jax.experimental.pallas import tpu as pltpu
from jax.experimental.pallas import tpu_sc as plsc

N, D, E = 10000, 128, 320000
NC, NS = 2, 16
DH = D // NC
EPT = E // NS
CH = 125
NCHUNK = EPT // CH
NB = 3
NP = 10240
RPT = NP // NS
BN_EPS = 1e-5


def _make_agg():
    mesh = plsc.VectorSubcoreMesh(core_axis_name="c", subcore_axis_name="s")

    @functools.partial(
        pl.kernel,
        mesh=mesh,
        compiler_params=pltpu.CompilerParams(use_tc_tiling_on_sc=False),
        out_type=jax.ShapeDtypeStruct((NP, D), jnp.float32),
        scratch_types=[
            pltpu.VMEM((NCHUNK, CH), jnp.int32),
            pltpu.VMEM((NCHUNK, CH), jnp.int32),
            [pltpu.VMEM((CH, DH), jnp.float32) for _ in range(NB)],
            pltpu.VMEM_SHARED((NP, DH), jnp.float32),
            [pltpu.SemaphoreType.DMA for _ in range(NB)],
            [pltpu.SemaphoreType.DMA for _ in range(NB)],
        ],
    )
    def agg(x_hbm, src_hbm, dst_hbm, zero_hbm, out_hbm,
            src_v, dst_v, rows, acc_sh, gsem, ssem):
        c = lax.axis_index("c")
        s = lax.axis_index("s")

        pltpu.sync_copy(zero_hbm, acc_sh.at[pl.ds(s * RPT, RPT)])

        pltpu.sync_copy(src_hbm.at[c, s], src_v)
        pltpu.sync_copy(dst_hbm.at[s], dst_v)
        plsc.subcore_barrier()

        def body(k, carry):
            for j in range(NB):
                m = NB * k + j
                bf = (j + 2) % NB

                @pl.when((m + 2 < NCHUNK) & (m >= 1))
                def _():
                    pltpu.make_async_copy(rows[bf], acc_sh.at[dst_v.at[0]],
                                          ssem[bf]).wait()

                @pl.when((m + 2 < NCHUNK) & (m + 2 >= 0))
                def _():
                    pltpu.async_copy(x_hbm.at[src_v.at[m + 2]], rows[bf],
                                     gsem[bf])

                @pl.when((m >= 0) & (m < NCHUNK))
                def _():
                    pltpu.make_async_copy(x_hbm.at[src_v.at[0]], rows[j],
                                          gsem[j]).wait()
                    pltpu.async_copy(rows[j], acc_sh.at[dst_v.at[m]],
                                     ssem[j], add=True)

            return carry

        lax.fori_loop(-1, (NCHUNK + NB - 1) // NB, body, 0)
        for j in range(NB):
            pltpu.make_async_copy(rows[j], acc_sh.at[dst_v.at[0]],
                                  ssem[j]).wait()
        plsc.subcore_barrier()

        pltpu.sync_copy(acc_sh.at[pl.ds(s * RPT, RPT)],
                        out_hbm.at[pl.ds(s * RPT, RPT), pl.ds(c * DH, DH)])

    return agg


_agg_cache = []


def _agg(*args):
    if not _agg_cache:
        _agg_cache.append(_make_agg())
    return _agg_cache[0](*args)


def _mlp1_body(parts_ref, x_ref, w_ref, b_ref, gm_ref, bt_ref, out_ref):
    aggv = parts_ref[:N] + x_ref[...]
    y = jnp.dot(aggv, w_ref[...], preferred_element_type=jnp.float32)
    y = y + b_ref[...]
    mu = jnp.mean(y, axis=0, keepdims=True)
    var = jnp.mean((y - mu) ** 2, axis=0, keepdims=True)
    yn = gm_ref[...] * (y - mu) * lax.rsqrt(var + BN_EPS) + bt_ref[...]
    out_ref[...] = jnp.maximum(yn, 0.0)


def _mlp2_body(parts_ref, x_ref, wa_ref, ba_ref, gm_ref, bt_ref, wb_ref,
               bb_ref, out_ref):
    aggv = parts_ref[:N] + x_ref[...]
    y = jnp.dot(aggv, wa_ref[...], preferred_element_type=jnp.float32)
    y = y + ba_ref[...]
    mu = jnp.mean(y, axis=0, keepdims=True)
    var = jnp.mean((y - mu) ** 2, axis=0, keepdims=True)
    z = jnp.maximum(gm_ref[...] * (y - mu) * lax.rsqrt(var + BN_EPS)
                    + bt_ref[...], 0.0)
    h2 = jnp.dot(z, wb_ref[...], preferred_element_type=jnp.float32)
    out_ref[...] = jnp.maximum(h2 + bb_ref[...], 0.0)


def _mlp1(parts, x, W1, b1, g1, be1):
    return pl.pallas_call(
        _mlp1_body,
        out_shape=jax.ShapeDtypeStruct((N, D), jnp.float32),
    )(parts, x, W1, b1.reshape(1, D), g1.reshape(1, D), be1.reshape(1, D))


def _mlp2(parts, x, W2a, b2a, g2, be2, W2b, b2b):
    return pl.pallas_call(
        _mlp2_body,
        out_shape=jax.ShapeDtypeStruct((N, D), jnp.float32),
    )(parts, x, W2a, b2a.reshape(1, D), g2.reshape(1, D), be2.reshape(1, D),
      W2b, b2b.reshape(1, D))


def kernel(g, h, W1, b1, g1, be1, W2a, b2a, g2, be2, W2b, b2b):
    base = 2 * g[0].astype(jnp.int32)
    src2 = jnp.stack([base, base + 1]).reshape(NC, NS, NCHUNK, CH)
    dst = g[1].astype(jnp.int32).reshape(NS, NCHUNK, CH)
    zeros = jnp.zeros((RPT, DH), jnp.float32)
    parts1 = _agg(h.reshape(N * NC, DH), src2, dst, zeros)
    h1 = _mlp1(parts1, h, W1, b1, g1, be1)
    parts2 = _agg(h1.reshape(N * NC, DH), src2, dst, zeros)
    return _mlp2(parts2, h1, W2a, b2a, g2, be2, W2b, b2b)


# speedup vs baseline: 1.6506x; 1.0118x over previous
import functools

import jax
import jax.numpy as jnp
from jax import lax
from jax.experimental import pallas as pl
from jax.experimental.pallas import tpu as pltpu
from jax.experimental.pallas import tpu_sc as plsc

N, D, E = 10000, 128, 320000
NC, NS = 2, 16
DH = D // NC
EPT = E // NS
CH = 125
NCHUNK = EPT // CH
NB = 3
NP = 10240
RPT = NP // NS
BN_EPS = 1e-5


def _make_agg():
    mesh = plsc.VectorSubcoreMesh(core_axis_name="c", subcore_axis_name="s")

    @functools.partial(
        pl.kernel,
        mesh=mesh,
        compiler_params=pltpu.CompilerParams(use_tc_tiling_on_sc=False),
        out_type=jax.ShapeDtypeStruct((NP, D), jnp.float32),
        scratch_types=[
            pltpu.VMEM((NCHUNK, CH), jnp.int32),
            pltpu.VMEM((NCHUNK, CH), jnp.int32),
            [pltpu.VMEM((CH, DH), jnp.float32) for _ in range(NB)],
            pltpu.VMEM_SHARED((NP, DH), jnp.float32),
            [pltpu.SemaphoreType.DMA for _ in range(NB)],
            [pltpu.SemaphoreType.DMA for _ in range(NB)],
        ],
    )
    def agg(x_hbm, src_hbm, dst_hbm, zero_hbm, out_hbm,
            src_v, dst_v, rows, acc_sh, gsem, ssem):
        c = lax.axis_index("c")
        s = lax.axis_index("s")

        pltpu.sync_copy(zero_hbm, acc_sh.at[pl.ds(s * RPT, RPT)])

        pltpu.sync_copy(src_hbm.at[s], src_v)
        pltpu.sync_copy(dst_hbm.at[s], dst_v)
        plsc.subcore_barrier()

        xc_hbm = x_hbm.at[pl.ds(c, NC * N - 1)]

        def body(k, carry):
            for j in range(NB):
                m = NB * k + j
                bf = (j + 2) % NB

                @pl.when((m + 2 < NCHUNK) & (m >= 1))
                def _():
                    pltpu.make_async_copy(rows[bf], acc_sh.at[dst_v.at[0]],
                                          ssem[bf]).wait()

                @pl.when((m + 2 < NCHUNK) & (m + 2 >= 0))
                def _():
                    pltpu.async_copy(xc_hbm.at[src_v.at[m + 2]], rows[bf],
                                     gsem[bf])

                @pl.when((m >= 0) & (m < NCHUNK))
                def _():
                    pltpu.make_async_copy(xc_hbm.at[src_v.at[0]], rows[j],
                                          gsem[j]).wait()
                    pltpu.async_copy(rows[j], acc_sh.at[dst_v.at[m]],
                                     ssem[j], add=True)

            return carry

        lax.fori_loop(-1, (NCHUNK + NB - 1) // NB, body, 0)
        for j in range(NB):
            pltpu.make_async_copy(rows[j], acc_sh.at[dst_v.at[0]],
                                  ssem[j]).wait()
        plsc.subcore_barrier()

        pltpu.sync_copy(acc_sh.at[pl.ds(s * RPT, RPT)],
                        out_hbm.at[pl.ds(s * RPT, RPT), pl.ds(c * DH, DH)])

    return agg


_agg_cache = []


def _agg(*args):
    if not _agg_cache:
        _agg_cache.append(_make_agg())
    return _agg_cache[0](*args)


def _mlp1_body(parts_ref, x_ref, w_ref, b_ref, gm_ref, bt_ref, out_ref):
    aggv = parts_ref[:N] + x_ref[...]
    y = jnp.dot(aggv, w_ref[...], preferred_element_type=jnp.float32)
    y = y + b_ref[...]
    mu = jnp.mean(y, axis=0, keepdims=True)
    var = jnp.mean((y - mu) ** 2, axis=0, keepdims=True)
    yn = gm_ref[...] * (y - mu) * lax.rsqrt(var + BN_EPS) + bt_ref[...]
    out_ref[...] = jnp.maximum(yn, 0.0)


def _mlp2_body(parts_ref, x_ref, wa_ref, ba_ref, gm_ref, bt_ref, wb_ref,
               bb_ref, out_ref):
    aggv = parts_ref[:N] + x_ref[...]
    y = jnp.dot(aggv, wa_ref[...], preferred_element_type=jnp.float32)
    y = y + ba_ref[...]
    mu = jnp.mean(y, axis=0, keepdims=True)
    var = jnp.mean((y - mu) ** 2, axis=0, keepdims=True)
    z = jnp.maximum(gm_ref[...] * (y - mu) * lax.rsqrt(var + BN_EPS)
                    + bt_ref[...], 0.0)
    h2 = jnp.dot(z, wb_ref[...], preferred_element_type=jnp.float32)
    out_ref[...] = jnp.maximum(h2 + bb_ref[...], 0.0)


def _mlp1(parts, x, W1, b1, g1, be1):
    return pl.pallas_call(
        _mlp1_body,
        out_shape=jax.ShapeDtypeStruct((N, D), jnp.float32),
    )(parts, x, W1, b1.reshape(1, D), g1.reshape(1, D), be1.reshape(1, D))


def _mlp2(parts, x, W2a, b2a, g2, be2, W2b, b2b):
    return pl.pallas_call(
        _mlp2_body,
        out_shape=jax.ShapeDtypeStruct((N, D), jnp.float32),
    )(parts, x, W2a, b2a.reshape(1, D), g2.reshape(1, D), be2.reshape(1, D),
      W2b, b2b.reshape(1, D))


def kernel(g, h, W1, b1, g1, be1, W2a, b2a, g2, be2, W2b, b2b):
    src2 = (2 * g[0].astype(jnp.int32)).reshape(NS, NCHUNK, CH)
    dst = g[1].astype(jnp.int32).reshape(NS, NCHUNK, CH)
    zeros = jnp.zeros((RPT, DH), jnp.float32)
    parts1 = _agg(h.reshape(N * NC, DH), src2, dst, zeros)
    h1 = _mlp1(parts1, h, W1, b1, g1, be1)
    parts2 = _agg(h1.reshape(N * NC, DH), src2, dst, zeros)
    return _mlp2(parts2, h1, W2a, b2a, g2, be2, W2b, b2b)


# speedup vs baseline: 1.6683x; 1.0107x over previous
import functools

import jax
import jax.numpy as jnp
from jax import lax
from jax.experimental import pallas as pl
from jax.experimental.pallas import tpu as pltpu
from jax.experimental.pallas import tpu_sc as plsc

N, D, E = 10000, 128, 320000
NC, NS = 2, 16
DH = D // NC
EPT = E // NS
CH = 125
NCHUNK = EPT // CH
NB = 3
NP = 10240
RPT = NP // NS
BN_EPS = 1e-5


def _make_agg():
    mesh = plsc.VectorSubcoreMesh(core_axis_name="c", subcore_axis_name="s")

    @functools.partial(
        pl.kernel,
        mesh=mesh,
        compiler_params=pltpu.CompilerParams(use_tc_tiling_on_sc=False),
        out_type=jax.ShapeDtypeStruct((NP, D), jnp.float32),
        scratch_types=[
            pltpu.VMEM((NCHUNK, CH), jnp.int32),
            pltpu.VMEM((NCHUNK, CH), jnp.int32),
            [pltpu.VMEM((CH, DH), jnp.float32) for _ in range(NB)],
            pltpu.VMEM_SHARED((NP, DH), jnp.float32),
            [pltpu.SemaphoreType.DMA for _ in range(NB)],
            [pltpu.SemaphoreType.DMA for _ in range(NB)],
        ],
    )
    def agg(x_hbm, src_hbm, dst_hbm, zero_hbm, out_hbm,
            src_v, dst_v, rows, acc_sh, gsem, ssem):
        c = lax.axis_index("c")
        s = lax.axis_index("s")

        pltpu.sync_copy(zero_hbm, acc_sh.at[pl.ds(s * RPT, RPT)])

        pltpu.sync_copy(src_hbm.at[s], src_v)
        pltpu.sync_copy(dst_hbm.at[s], dst_v)
        plsc.subcore_barrier()

        xc_hbm = x_hbm.at[pl.ds(c, NC * N - 1)]

        def body(k, carry):
            for j in range(NB):
                m = NB * k + j
                bf = (j + 2) % NB

                @pl.when((m + 2 < NCHUNK) & (m >= 1))
                def _():
                    pltpu.make_async_copy(rows[bf], acc_sh.at[dst_v.at[0]],
                                          ssem[bf]).wait()

                @pl.when((m + 2 < NCHUNK) & (m + 2 >= 0))
                def _():
                    pltpu.async_copy(xc_hbm.at[src_v.at[m + 2]], rows[bf],
                                     gsem[bf])

                @pl.when((m >= 0) & (m < NCHUNK))
                def _():
                    pltpu.make_async_copy(xc_hbm.at[src_v.at[0]], rows[j],
                                          gsem[j]).wait()
                    pltpu.async_copy(rows[j], acc_sh.at[dst_v.at[m]],
                                     ssem[j], add=True)

            return carry

        lax.fori_loop(-1, (NCHUNK + NB - 1) // NB, body, 0)
        for j in range(NB):
            pltpu.make_async_copy(rows[j], acc_sh.at[dst_v.at[0]],
                                  ssem[j]).wait()
        plsc.subcore_barrier()

        pltpu.sync_copy(acc_sh.at[pl.ds(s * RPT, RPT)],
                        out_hbm.at[pl.ds(s * RPT, RPT), pl.ds(c * DH, DH)])

    return agg


_agg_cache = []


def _agg(*args):
    if not _agg_cache:
        _agg_cache.append(_make_agg())
    return _agg_cache[0](*args)


def _mlp1_body(parts_ref, x_ref, w_ref, b_ref, gm_ref, bt_ref, out_ref):
    aggv = parts_ref[:N] + x_ref[...]
    y = jnp.dot(aggv, w_ref[...], preferred_element_type=jnp.float32)
    y = y + b_ref[...]
    mu = jnp.mean(y, axis=0, keepdims=True)
    var = jnp.mean((y - mu) ** 2, axis=0, keepdims=True)
    yn = gm_ref[...] * (y - mu) * lax.rsqrt(var + BN_EPS) + bt_ref[...]
    out_ref[...] = jnp.maximum(yn, 0.0)


def _mlp2_body(parts_ref, x_ref, wa_ref, ba_ref, gm_ref, bt_ref, wb_ref,
               bb_ref, out_ref):
    aggv = parts_ref[:N] + x_ref[...]
    y = jnp.dot(aggv, wa_ref[...], preferred_element_type=jnp.float32)
    y = y + ba_ref[...]
    mu = jnp.mean(y, axis=0, keepdims=True)
    var = jnp.mean((y - mu) ** 2, axis=0, keepdims=True)
    z = jnp.maximum(gm_ref[...] * (y - mu) * lax.rsqrt(var + BN_EPS)
                    + bt_ref[...], 0.0)
    h2 = jnp.dot(z, wb_ref[...], preferred_element_type=jnp.float32)
    out_ref[...] = jnp.maximum(h2 + bb_ref[...], 0.0)


def _mlp1(parts, x, W1, b1, g1, be1):
    return pl.pallas_call(
        _mlp1_body,
        out_shape=jax.ShapeDtypeStruct((N, D), jnp.float32),
    )(parts, x, W1, b1.reshape(1, D), g1.reshape(1, D), be1.reshape(1, D))


def _mlp2(parts, x, W2a, b2a, g2, be2, W2b, b2b):
    return pl.pallas_call(
        _mlp2_body,
        out_shape=jax.ShapeDtypeStruct((N, D), jnp.float32),
    )(parts, x, W2a, b2a.reshape(1, D), g2.reshape(1, D), be2.reshape(1, D),
      W2b, b2b.reshape(1, D))


def kernel(g, h, W1, b1, g1, be1, W2a, b2a, g2, be2, W2b, b2b):
    src0, dst0 = lax.optimization_barrier(
        (g[0].astype(jnp.int32), g[1].astype(jnp.int32)))
    src2 = (2 * src0).reshape(NS, NCHUNK, CH)
    dst = dst0.reshape(NS, NCHUNK, CH)
    zeros = jnp.zeros((RPT, DH), jnp.float32)
    parts1 = _agg(h.reshape(N * NC, DH), src2, dst, zeros)
    h1 = _mlp1(parts1, h, W1, b1, g1, be1)
    parts2 = _agg(h1.reshape(N * NC, DH), src2, dst, zeros)
    return _mlp2(parts2, h1, W2a, b2a, g2, be2, W2b, b2b)


# speedup vs baseline: 1.6893x; 1.0126x over previous
import functools

import jax
import jax.numpy as jnp
from jax import lax
from jax.experimental import pallas as pl
from jax.experimental.pallas import tpu as pltpu
from jax.experimental.pallas import tpu_sc as plsc

N, D, E = 10000, 128, 320000
NC, NS = 2, 16
DH = D // NC
EPT = E // NS
CH = 125
NCHUNK = EPT // CH
NB = 3
NP = 10240
RPT = NP // NS
BN_EPS = 1e-5


def _make_agg():
    mesh = plsc.VectorSubcoreMesh(core_axis_name="c", subcore_axis_name="s")

    @functools.partial(
        pl.kernel,
        mesh=mesh,
        compiler_params=pltpu.CompilerParams(use_tc_tiling_on_sc=False),
        out_type=jax.ShapeDtypeStruct((NP, D), jnp.float32),
        scratch_types=[
            pltpu.VMEM((NCHUNK, CH), jnp.int32),
            pltpu.VMEM((NCHUNK, CH), jnp.int32),
            [pltpu.VMEM((CH, DH), jnp.float32) for _ in range(NB)],
            pltpu.VMEM_SHARED((NP, DH), jnp.float32),
            [pltpu.SemaphoreType.DMA for _ in range(NB)],
            [pltpu.SemaphoreType.DMA for _ in range(NB)],
        ],
    )
    def agg(x_hbm, src_hbm, dst_hbm, zero_hbm, out_hbm,
            src_v, dst_v, rows, acc_sh, gsem, ssem):
        c = lax.axis_index("c")
        s = lax.axis_index("s")

        z0 = pltpu.async_copy(zero_hbm, acc_sh.at[pl.ds(s * RPT, RPT)],
                              gsem[0])
        i0 = pltpu.async_copy(src_hbm.at[s], src_v, gsem[1])
        i1 = pltpu.async_copy(dst_hbm.at[s], dst_v, gsem[2])
        z0.wait()
        i0.wait()
        i1.wait()
        plsc.subcore_barrier()

        xc_hbm = x_hbm.at[pl.ds(c, NC * N - 1)]

        def body(k, carry):
            for j in range(NB):
                m = NB * k + j
                bf = (j + 2) % NB

                @pl.when((m + 2 < NCHUNK) & (m >= 1))
                def _():
                    pltpu.make_async_copy(rows[bf], acc_sh.at[dst_v.at[0]],
                                          ssem[bf]).wait()

                @pl.when((m + 2 < NCHUNK) & (m + 2 >= 0))
                def _():
                    pltpu.async_copy(xc_hbm.at[src_v.at[m + 2]], rows[bf],
                                     gsem[bf])

                @pl.when((m >= 0) & (m < NCHUNK))
                def _():
                    pltpu.make_async_copy(xc_hbm.at[src_v.at[0]], rows[j],
                                          gsem[j]).wait()
                    pltpu.async_copy(rows[j], acc_sh.at[dst_v.at[m]],
                                     ssem[j], add=True)

            return carry

        lax.fori_loop(-1, (NCHUNK + NB - 1) // NB, body, 0)
        for j in range(NB):
            pltpu.make_async_copy(rows[j], acc_sh.at[dst_v.at[0]],
                                  ssem[j]).wait()
        plsc.subcore_barrier()

        pltpu.sync_copy(acc_sh.at[pl.ds(s * RPT, RPT)],
                        out_hbm.at[pl.ds(s * RPT, RPT), pl.ds(c * DH, DH)])

    return agg


_agg_cache = []


def _agg(*args):
    if not _agg_cache:
        _agg_cache.append(_make_agg())
    return _agg_cache[0](*args)


def _mlp1_body(parts_ref, x_ref, w_ref, b_ref, gm_ref, bt_ref, out_ref):
    aggv = parts_ref[:N] + x_ref[...]
    y = jnp.dot(aggv, w_ref[...], preferred_element_type=jnp.float32)
    y = y + b_ref[...]
    mu = jnp.mean(y, axis=0, keepdims=True)
    var = jnp.mean((y - mu) ** 2, axis=0, keepdims=True)
    yn = gm_ref[...] * (y - mu) * lax.rsqrt(var + BN_EPS) + bt_ref[...]
    out_ref[...] = jnp.maximum(yn, 0.0)


def _mlp2_body(parts_ref, x_ref, wa_ref, ba_ref, gm_ref, bt_ref, wb_ref,
               bb_ref, out_ref):
    aggv = parts_ref[:N] + x_ref[...]
    y = jnp.dot(aggv, wa_ref[...], preferred_element_type=jnp.float32)
    y = y + ba_ref[...]
    mu = jnp.mean(y, axis=0, keepdims=True)
    var = jnp.mean((y - mu) ** 2, axis=0, keepdims=True)
    z = jnp.maximum(gm_ref[...] * (y - mu) * lax.rsqrt(var + BN_EPS)
                    + bt_ref[...], 0.0)
    h2 = jnp.dot(z, wb_ref[...], preferred_element_type=jnp.float32)
    out_ref[...] = jnp.maximum(h2 + bb_ref[...], 0.0)


def _mlp1(parts, x, W1, b1, g1, be1):
    return pl.pallas_call(
        _mlp1_body,
        out_shape=jax.ShapeDtypeStruct((N, D), jnp.float32),
    )(parts, x, W1, b1.reshape(1, D), g1.reshape(1, D), be1.reshape(1, D))


def _mlp2(parts, x, W2a, b2a, g2, be2, W2b, b2b):
    return pl.pallas_call(
        _mlp2_body,
        out_shape=jax.ShapeDtypeStruct((N, D), jnp.float32),
    )(parts, x, W2a, b2a.reshape(1, D), g2.reshape(1, D), be2.reshape(1, D),
      W2b, b2b.reshape(1, D))


def kernel(g, h, W1, b1, g1, be1, W2a, b2a, g2, be2, W2b, b2b):
    src0, dst0 = lax.optimization_barrier(
        (g[0].astype(jnp.int32), g[1].astype(jnp.int32)))
    src2 = (2 * src0).reshape(NS, NCHUNK, CH)
    dst = dst0.reshape(NS, NCHUNK, CH)
    zeros = jnp.zeros((RPT, DH), jnp.float32)
    parts1 = _agg(h.reshape(N * NC, DH), src2, dst, zeros)
    h1 = _mlp1(parts1, h, W1, b1, g1, be1)
    parts2 = _agg(h1.reshape(N * NC, DH), src2, dst, zeros)
    return _mlp2(parts2, h1, W2a, b2a, g2, be2, W2b, b2b)
